# Initial kernel scaffold; baseline (speedup 1.0000x reference)
import jax
import jax.numpy as jnp
from jax.experimental import pallas as pl


def kernel(feat, edge_index, node_types, W_shared, b_shared, W2, bn_gamma, bn_beta, bn_mean, bn_var, W_het, W_self_shared, b_self_shared, W_self, bias_out):
    raise NotImplementedError("write your pallas kernel here")


# speedup vs baseline: 16.6147x; 16.6147x over previous
import functools

import jax
import jax.numpy as jnp
from jax import lax
from jax.experimental import pallas as pl
from jax.experimental.pallas import tpu as pltpu
from jax.experimental.pallas import tpu_sc as plsc

N = 10000
E = 320000
D = 128
NC = 2
NS = 16
L = 16

EPW = E // NS
N_PAD = 10240
RPW = N_PAD // NS
CH = 128
NFULL = EPW // CH
TAIL = EPW - NFULL * CH
EPC2 = E // NC
EPW2 = EPC2 // NS
NF2 = EPW2 // CH
TAIL2 = EPW2 - NF2 * CH

BN_ROWS = 1000


def _prep_body(feat_ref, ntf_ref, Wsh_ref, bsh_ref, W2_ref, scale_ref,
               shift_ref, Whet_ref, Wss_ref, bss_ref, Wself_ref,
               q0_ref, q1_ref, hself_ref):
    f = feat_ref[...]
    t = jnp.dot(f, Wsh_ref[...], preferred_element_type=jnp.float32) + bsh_ref[...]
    h2 = jnp.dot(t, W2_ref[...], preferred_element_type=jnp.float32)
    h = jnp.maximum(h2 * scale_ref[...] + shift_ref[...], 0.0)
    m0 = ntf_ref[...] == 0.0
    p0 = jnp.dot(h, Whet_ref[0], preferred_element_type=jnp.float32)
    p1 = jnp.dot(h, Whet_ref[1], preferred_element_type=jnp.float32)
    p2 = jnp.dot(h, Whet_ref[2], preferred_element_type=jnp.float32)
    p3 = jnp.dot(h, Whet_ref[3], preferred_element_type=jnp.float32)
    q0_ref[...] = jnp.where(m0, p0, p2)
    q1_ref[...] = jnp.where(m0, p1, p3)
    hs = jnp.dot(f, Wss_ref[...], preferred_element_type=jnp.float32) + bss_ref[...]
    s0 = jnp.dot(hs, Wself_ref[0], preferred_element_type=jnp.float32)
    s1 = jnp.dot(hs, Wself_ref[1], preferred_element_type=jnp.float32)
    hself_ref[...] = jnp.where(m0, s0, s1)


def _prep(feat, ntf, W_shared, b_shared, W2, scale, shift, W_het,
          W_self_shared, b_self_shared, W_self):
    nblk = N // BN_ROWS
    full2 = pl.BlockSpec((128, 128), lambda i: (0, 0))
    row1 = pl.BlockSpec((1, 128), lambda i: (0, 0))
    return pl.pallas_call(
        _prep_body,
        grid=(nblk,),
        in_specs=[
            pl.BlockSpec((BN_ROWS, 128), lambda i: (i, 0)),
            pl.BlockSpec((BN_ROWS, 1), lambda i: (i, 0)),
            full2, row1, full2, row1, row1,
            pl.BlockSpec((4, 128, 128), lambda i: (0, 0, 0)),
            full2, row1,
            pl.BlockSpec((2, 128, 128), lambda i: (0, 0, 0)),
        ],
        out_specs=[
            pl.BlockSpec((BN_ROWS, 128), lambda i: (i, 0)),
            pl.BlockSpec((BN_ROWS, 128), lambda i: (i, 0)),
            pl.BlockSpec((BN_ROWS, 128), lambda i: (i, 0)),
        ],
        out_shape=[
            jax.ShapeDtypeStruct((N, 128), jnp.float32),
            jax.ShapeDtypeStruct((N, 128), jnp.float32),
            jax.ShapeDtypeStruct((N, 128), jnp.float32),
        ],
    )(feat, ntf, W_shared, b_shared, W2, scale, shift, W_het,
      W_self_shared, b_self_shared, W_self)


def _sc_agg(src, dst, q0, q1, zA, ones_rows):
    mesh = plsc.VectorSubcoreMesh(core_axis_name="c", subcore_axis_name="s")

    @functools.partial(
        pl.kernel,
        mesh=mesh,
        out_type=[
            jax.ShapeDtypeStruct((NC, N_PAD, D), jnp.float32),
            jax.ShapeDtypeStruct((NC, N_PAD, D), jnp.float32),
        ],
        scratch_types=[
            pltpu.VMEM((CH,), jnp.int32),
            pltpu.VMEM((CH,), jnp.int32),
            pltpu.VMEM((CH, D), jnp.float32),
            pltpu.VMEM((TAIL,), jnp.int32),
            pltpu.VMEM((TAIL,), jnp.int32),
            pltpu.VMEM((TAIL, D), jnp.float32),
            pltpu.VMEM((TAIL2,), jnp.int32),
            pltpu.VMEM_SHARED((N_PAD, D), jnp.float32),
            pltpu.SemaphoreType.DMA,
        ],
    )
    def agg(src_hbm, dst_hbm, q0_hbm, q1_hbm, zA_hbm, ones_hbm,
            outA_hbm, outD_hbm,
            srcc_v, dstc_v, rows_v, srcc_t, dstc_t, rows_t, dstc_t2,
            A_sp, sem):
        cid = lax.axis_index("c")
        sid = lax.axis_index("s")
        rbase = sid * RPW

        def zero_acc():
            for b in range(RPW // CH):
                rb = rbase + b * CH
                pltpu.sync_copy(zA_hbm.at[pl.ds(rb, CH)], rows_v)
                pltpu.sync_copy(rows_v, A_sp.at[pl.ds(rb, CH)])

        def export(dst_out):
            for b in range(RPW // CH):
                rb = rbase + b * CH
                pltpu.sync_copy(A_sp.at[pl.ds(rb, CH)], rows_v)
                pltpu.sync_copy(rows_v, dst_out.at[cid, pl.ds(rb, CH)])

        zero_acc()
        plsc.subcore_barrier()

        ebase1 = sid * EPW

        def do_chunk1(q_hbm, cbase, n, srcc_ref, dstc_ref, rows_ref):
            pltpu.sync_copy(dst_hbm.at[pl.ds(ebase1 + cbase, n)], dstc_ref)
            pltpu.sync_copy(src_hbm.at[pl.ds(ebase1 + cbase, n)], srcc_ref)
            pltpu.async_copy(q_hbm.at[srcc_ref], rows_ref, sem).wait()
            pltpu.sync_copy(rows_ref, A_sp.at[dstc_ref], add=True)

        def sweep(q_hbm):
            def chunk_body(i, c):
                do_chunk1(q_hbm, i * CH, CH, srcc_v, dstc_v, rows_v)
                return c
            lax.fori_loop(0, NFULL, chunk_body, 0)
            do_chunk1(q_hbm, NFULL * CH, TAIL, srcc_t, dstc_t, rows_t)

        @pl.when(cid == 0)
        def _():
            sweep(q0_hbm)

        @pl.when(cid == 1)
        def _():
            sweep(q1_hbm)

        plsc.subcore_barrier()
        export(outA_hbm)
        plsc.subcore_barrier()

        zero_acc()
        pltpu.sync_copy(ones_hbm, rows_v)
        plsc.subcore_barrier()

        ebase2 = cid * EPC2 + sid * EPW2

        def deg_chunk(i, c):
            pltpu.sync_copy(dst_hbm.at[pl.ds(ebase2 + i * CH, CH)], dstc_v)
            pltpu.sync_copy(rows_v, A_sp.at[dstc_v], add=True)
            return c
        lax.fori_loop(0, NF2, deg_chunk, 0)
        pltpu.sync_copy(dst_hbm.at[pl.ds(ebase2 + NF2 * CH, TAIL2)], dstc_t2)
        pltpu.sync_copy(rows_v.at[pl.ds(0, TAIL2)], A_sp.at[dstc_t2], add=True)

        plsc.subcore_barrier()
        export(outD_hbm)

    return agg(src, dst, q0, q1, zA, ones_rows)


def _combine_body(A_ref, deg_ref, hself_ref, ntf_ref, bias_ref, out_ref):
    a = jnp.where(ntf_ref[...] == 0.0, A_ref[0], A_ref[1])
    d = deg_ref[0, :, 0:1] + deg_ref[1, :, 0:1]
    out_ref[...] = hself_ref[...] + a / jnp.maximum(d, 1.0) + bias_ref[...]


def _combine(A_part, deg_part, h_self, ntf, bias_row):
    nblk = N // BN_ROWS
    return pl.pallas_call(
        _combine_body,
        grid=(nblk,),
        in_specs=[
            pl.BlockSpec((2, BN_ROWS, 128), lambda i: (0, i, 0)),
            pl.BlockSpec((2, BN_ROWS, 128), lambda i: (0, i, 0)),
            pl.BlockSpec((BN_ROWS, 128), lambda i: (i, 0)),
            pl.BlockSpec((BN_ROWS, 1), lambda i: (i, 0)),
            pl.BlockSpec((1, 128), lambda i: (0, 0)),
        ],
        out_specs=pl.BlockSpec((BN_ROWS, 128), lambda i: (i, 0)),
        out_shape=jax.ShapeDtypeStruct((N, 128), jnp.float32),
    )(A_part, deg_part, h_self, ntf, bias_row)


def kernel(feat, edge_index, node_types, W_shared, b_shared, W2, bn_gamma,
           bn_beta, bn_mean, bn_var, W_het, W_self_shared, b_self_shared,
           W_self, bias_out):
    scale = (bn_gamma / jnp.sqrt(bn_var + 1e-5)).reshape(1, D)
    shift = (bn_beta - bn_mean * (bn_gamma / jnp.sqrt(bn_var + 1e-5))).reshape(1, D)
    ntf = node_types.astype(jnp.float32).reshape(N, 1)

    q0, q1, h_self = _prep(feat, ntf, W_shared, b_shared.reshape(1, D), W2,
                           scale, shift, W_het, W_self_shared,
                           b_self_shared.reshape(1, D), W_self)

    src = edge_index[0]
    dst = edge_index[1]
    zA = jnp.zeros((N_PAD, D), jnp.float32)
    ones_rows = jnp.ones((CH, D), jnp.float32)
    A_part, deg_part = _sc_agg(src, dst, q0, q1, zA, ones_rows)

    return _combine(A_part, deg_part, h_self, ntf, bias_out.reshape(1, D))


# speedup vs baseline: 25.9847x; 1.5640x over previous
import functools

import jax
import jax.numpy as jnp
from jax import lax
from jax.experimental import pallas as pl
from jax.experimental.pallas import tpu as pltpu
from jax.experimental.pallas import tpu_sc as plsc

N = 10000
E = 320000
D = 128
NC = 2
NS = 16
L = 16

N_PAD = 10240
RPW = N_PAD // NS
CH = 128
NCHUNK = E // CH
P1_BASE = NCHUNK // NS
P1_XTRA = NCHUNK - P1_BASE * NS
SB1 = 12
NSB1 = P1_BASE // SB1
P2_CORE = NCHUNK // NC
P2_BASE = P2_CORE // NS
P2_XTRA = P2_CORE - P2_BASE * NS
SB2 = 13
NSB2 = P2_BASE // SB2

BN_ROWS = 1000


def _prep_body(feat_ref, ntf_ref, Wsh_ref, bsh_ref, W2_ref, scale_ref,
               shift_ref, Whet_ref, Wss_ref, bss_ref, Wself_ref,
               q0_ref, q1_ref, hself_ref):
    f = feat_ref[...]
    t = jnp.dot(f, Wsh_ref[...], preferred_element_type=jnp.float32) + bsh_ref[...]
    h2 = jnp.dot(t, W2_ref[...], preferred_element_type=jnp.float32)
    h = jnp.maximum(h2 * scale_ref[...] + shift_ref[...], 0.0)
    m0 = ntf_ref[...] == 0.0
    p0 = jnp.dot(h, Whet_ref[0], preferred_element_type=jnp.float32)
    p1 = jnp.dot(h, Whet_ref[1], preferred_element_type=jnp.float32)
    p2 = jnp.dot(h, Whet_ref[2], preferred_element_type=jnp.float32)
    p3 = jnp.dot(h, Whet_ref[3], preferred_element_type=jnp.float32)
    q0_ref[...] = jnp.where(m0, p0, p2)
    q1_ref[...] = jnp.where(m0, p1, p3)
    hs = jnp.dot(f, Wss_ref[...], preferred_element_type=jnp.float32) + bss_ref[...]
    s0 = jnp.dot(hs, Wself_ref[0], preferred_element_type=jnp.float32)
    s1 = jnp.dot(hs, Wself_ref[1], preferred_element_type=jnp.float32)
    hself_ref[...] = jnp.where(m0, s0, s1)


def _prep(feat, ntf, W_shared, b_shared, W2, scale, shift, W_het,
          W_self_shared, b_self_shared, W_self):
    nblk = N // BN_ROWS
    full2 = pl.BlockSpec((128, 128), lambda i: (0, 0))
    row1 = pl.BlockSpec((1, 128), lambda i: (0, 0))
    return pl.pallas_call(
        _prep_body,
        grid=(nblk,),
        in_specs=[
            pl.BlockSpec((BN_ROWS, 128), lambda i: (i, 0)),
            pl.BlockSpec((BN_ROWS, 1), lambda i: (i, 0)),
            full2, row1, full2, row1, row1,
            pl.BlockSpec((4, 128, 128), lambda i: (0, 0, 0)),
            full2, row1,
            pl.BlockSpec((2, 128, 128), lambda i: (0, 0, 0)),
        ],
        out_specs=[
            pl.BlockSpec((BN_ROWS, 128), lambda i: (i, 0)),
            pl.BlockSpec((BN_ROWS, 128), lambda i: (i, 0)),
            pl.BlockSpec((BN_ROWS, 128), lambda i: (i, 0)),
        ],
        out_shape=[
            jax.ShapeDtypeStruct((N, 128), jnp.float32),
            jax.ShapeDtypeStruct((N, 128), jnp.float32),
            jax.ShapeDtypeStruct((N, 128), jnp.float32),
        ],
    )(feat, ntf, W_shared, b_shared, W2, scale, shift, W_het,
      W_self_shared, b_self_shared, W_self)


def _sc_agg(src, dst, q0, q1, zA, ones_rows):
    mesh = plsc.VectorSubcoreMesh(core_axis_name="c", subcore_axis_name="s")

    @functools.partial(
        pl.kernel,
        mesh=mesh,
        out_type=[
            jax.ShapeDtypeStruct((NC, N_PAD, D), jnp.float32),
            jax.ShapeDtypeStruct((NC, N_PAD, D), jnp.float32),
        ],
        scratch_types=[
            pltpu.VMEM((SB2 * CH,), jnp.int32),
            pltpu.VMEM((SB2 * CH,), jnp.int32),
            pltpu.VMEM((CH,), jnp.int32),
            pltpu.VMEM((CH,), jnp.int32),
            pltpu.VMEM((CH, D), jnp.float32),
            pltpu.VMEM((CH, D), jnp.float32),
            pltpu.VMEM_SHARED((N_PAD, D), jnp.float32),
            pltpu.SemaphoreType.DMA,
            pltpu.SemaphoreType.DMA,
        ],
    )
    def agg(src_hbm, dst_hbm, q0_hbm, q1_hbm, zA_hbm, ones_hbm,
            outA_hbm, outD_hbm,
            srcb_v, dstb_v, dstc_v, srcc_v, rows_a, rows_b, A_sp,
            sem_a, sem_b):
        cid = lax.axis_index("c")
        sid = lax.axis_index("s")
        rbase = sid * RPW

        def zero_acc():
            for b in range(RPW // CH):
                rb = rbase + b * CH
                pltpu.sync_copy(zA_hbm.at[pl.ds(rb, CH)], rows_a)
                pltpu.sync_copy(rows_a, A_sp.at[pl.ds(rb, CH)])

        def export(dst_out):
            for b in range(RPW // CH):
                rb = rbase + b * CH
                pltpu.sync_copy(A_sp.at[pl.ds(rb, CH)], rows_a)
                pltpu.sync_copy(rows_a, dst_out.at[cid, pl.ds(rb, CH)])

        def load_dstc(j):
            for k in range(CH // L):
                dstc_v[pl.ds(k * L, L)] = dstb_v[pl.ds(j * CH + k * L, L)]

        zero_acc()
        plsc.subcore_barrier()

        cb1 = P1_BASE * sid + jnp.minimum(sid, P1_XTRA)

        def sweep(q_hbm):
            def sblock(sb, c):
                crow = cb1 + sb * SB1
                eb = crow * CH
                pltpu.sync_copy(src_hbm.at[pl.ds(eb, SB1 * CH)],
                                srcb_v.at[pl.ds(0, SB1 * CH)])
                pltpu.sync_copy(dst_hbm.at[pl.ds(eb, SB1 * CH)],
                                dstb_v.at[pl.ds(0, SB1 * CH)])
                cp = pltpu.async_copy(
                    q_hbm.at[srcb_v.at[pl.ds(0, CH)]], rows_a, sem_a)
                for j in range(SB1):
                    cur = rows_a if j % 2 == 0 else rows_b
                    nxt = rows_b if j % 2 == 0 else rows_a
                    nsem = sem_b if j % 2 == 0 else sem_a
                    load_dstc(j)
                    cp.wait()
                    if j + 1 < SB1:
                        cp = pltpu.async_copy(
                            q_hbm.at[srcb_v.at[pl.ds((j + 1) * CH, CH)]],
                            nxt, nsem)
                    pltpu.sync_copy(cur, A_sp.at[dstc_v], add=True)
                return c
            lax.fori_loop(0, NSB1, sblock, 0)

            @pl.when(sid < P1_XTRA)
            def _():
                eb = (cb1 + P1_BASE) * CH
                pltpu.sync_copy(src_hbm.at[pl.ds(eb, CH)], srcc_v)
                pltpu.sync_copy(dst_hbm.at[pl.ds(eb, CH)], dstc_v)
                pltpu.async_copy(q_hbm.at[srcc_v], rows_a, sem_a).wait()
                pltpu.sync_copy(rows_a, A_sp.at[dstc_v], add=True)

        @pl.when(cid == 0)
        def _():
            sweep(q0_hbm)

        @pl.when(cid == 1)
        def _():
            sweep(q1_hbm)

        plsc.subcore_barrier()
        export(outA_hbm)
        plsc.subcore_barrier()

        zero_acc()
        pltpu.sync_copy(ones_hbm, rows_b)
        plsc.subcore_barrier()

        cb2 = P2_CORE * cid + P2_BASE * sid + jnp.minimum(sid, P2_XTRA)

        def sblock2(sb, c):
            eb = (cb2 + sb * SB2) * CH
            pltpu.sync_copy(dst_hbm.at[pl.ds(eb, SB2 * CH)], dstb_v)
            for j in range(SB2):
                load_dstc(j)
                pltpu.sync_copy(rows_b, A_sp.at[dstc_v], add=True)
            return c
        lax.fori_loop(0, NSB2, sblock2, 0)

        @pl.when(sid < P2_XTRA)
        def _():
            eb = (cb2 + P2_BASE) * CH
            pltpu.sync_copy(dst_hbm.at[pl.ds(eb, CH)], dstc_v)
            pltpu.sync_copy(rows_b, A_sp.at[dstc_v], add=True)

        plsc.subcore_barrier()
        export(outD_hbm)

    return agg(src, dst, q0, q1, zA, ones_rows)


def _combine_body(A_ref, deg_ref, hself_ref, ntf_ref, bias_ref, out_ref):
    a = jnp.where(ntf_ref[...] == 0.0, A_ref[0], A_ref[1])
    d = deg_ref[0, :, 0:1] + deg_ref[1, :, 0:1]
    out_ref[...] = hself_ref[...] + a / jnp.maximum(d, 1.0) + bias_ref[...]


def _combine(A_part, deg_part, h_self, ntf, bias_row):
    nblk = N // BN_ROWS
    return pl.pallas_call(
        _combine_body,
        grid=(nblk,),
        in_specs=[
            pl.BlockSpec((2, BN_ROWS, 128), lambda i: (0, i, 0)),
            pl.BlockSpec((2, BN_ROWS, 128), lambda i: (0, i, 0)),
            pl.BlockSpec((BN_ROWS, 128), lambda i: (i, 0)),
            pl.BlockSpec((BN_ROWS, 1), lambda i: (i, 0)),
            pl.BlockSpec((1, 128), lambda i: (0, 0)),
        ],
        out_specs=pl.BlockSpec((BN_ROWS, 128), lambda i: (i, 0)),
        out_shape=jax.ShapeDtypeStruct((N, 128), jnp.float32),
    )(A_part, deg_part, h_self, ntf, bias_row)


def kernel(feat, edge_index, node_types, W_shared, b_shared, W2, bn_gamma,
           bn_beta, bn_mean, bn_var, W_het, W_self_shared, b_self_shared,
           W_self, bias_out):
    scale = (bn_gamma / jnp.sqrt(bn_var + 1e-5)).reshape(1, D)
    shift = (bn_beta - bn_mean * (bn_gamma / jnp.sqrt(bn_var + 1e-5))).reshape(1, D)
    ntf = node_types.astype(jnp.float32).reshape(N, 1)

    q0, q1, h_self = _prep(feat, ntf, W_shared, b_shared.reshape(1, D), W2,
                           scale, shift, W_het, W_self_shared,
                           b_self_shared.reshape(1, D), W_self)

    src = edge_index[0]
    dst = edge_index[1]
    zA = jnp.zeros((N_PAD, D), jnp.float32)
    ones_rows = jnp.ones((CH, D), jnp.float32)
    A_part, deg_part = _sc_agg(src, dst, q0, q1, zA, ones_rows)

    return _combine(A_part, deg_part, h_self, ntf, bias_out.reshape(1, D))


# speedup vs baseline: 26.0308x; 1.0018x over previous
import functools

import jax
import jax.numpy as jnp
from jax import lax
from jax.experimental import pallas as pl
from jax.experimental.pallas import tpu as pltpu
from jax.experimental.pallas import tpu_sc as plsc

N = 10000
E = 320000
D = 128
NC = 2
NS = 16
L = 16

N_PAD = 10240
RPW = N_PAD // NS
CH = 128
NCHUNK = E // CH
P1_BASE = NCHUNK // NS
P1_XTRA = NCHUNK - P1_BASE * NS
SB1 = 12
NSB1 = P1_BASE // SB1
P2_CORE = NCHUNK // NC
P2_BASE = P2_CORE // NS
P2_XTRA = P2_CORE - P2_BASE * NS
SB2 = 13
NSB2 = P2_BASE // SB2

BN_ROWS = 1000


def _prep_body(feat_ref, ntf_ref, Wsh_ref, bsh_ref, W2_ref, scale_ref,
               shift_ref, Whet_ref, Wss_ref, bss_ref, Wself_ref,
               q0_ref, q1_ref, hself_ref):
    f = feat_ref[...]
    t = jnp.dot(f, Wsh_ref[...], preferred_element_type=jnp.float32) + bsh_ref[...]
    h2 = jnp.dot(t, W2_ref[...], preferred_element_type=jnp.float32)
    h = jnp.maximum(h2 * scale_ref[...] + shift_ref[...], 0.0)
    m0 = ntf_ref[...] == 0.0
    p0 = jnp.dot(h, Whet_ref[0], preferred_element_type=jnp.float32)
    p1 = jnp.dot(h, Whet_ref[1], preferred_element_type=jnp.float32)
    p2 = jnp.dot(h, Whet_ref[2], preferred_element_type=jnp.float32)
    p3 = jnp.dot(h, Whet_ref[3], preferred_element_type=jnp.float32)
    q0_ref[...] = jnp.where(m0, p0, p2)
    q1_ref[...] = jnp.where(m0, p1, p3)
    hs = jnp.dot(f, Wss_ref[...], preferred_element_type=jnp.float32) + bss_ref[...]
    s0 = jnp.dot(hs, Wself_ref[0], preferred_element_type=jnp.float32)
    s1 = jnp.dot(hs, Wself_ref[1], preferred_element_type=jnp.float32)
    hself_ref[...] = jnp.where(m0, s0, s1)


def _prep(feat, ntf, W_shared, b_shared, W2, scale, shift, W_het,
          W_self_shared, b_self_shared, W_self):
    nblk = N // BN_ROWS
    full2 = pl.BlockSpec((128, 128), lambda i: (0, 0))
    row1 = pl.BlockSpec((1, 128), lambda i: (0, 0))
    return pl.pallas_call(
        _prep_body,
        grid=(nblk,),
        in_specs=[
            pl.BlockSpec((BN_ROWS, 128), lambda i: (i, 0)),
            pl.BlockSpec((BN_ROWS, 1), lambda i: (i, 0)),
            full2, row1, full2, row1, row1,
            pl.BlockSpec((4, 128, 128), lambda i: (0, 0, 0)),
            full2, row1,
            pl.BlockSpec((2, 128, 128), lambda i: (0, 0, 0)),
        ],
        out_specs=[
            pl.BlockSpec((BN_ROWS, 128), lambda i: (i, 0)),
            pl.BlockSpec((BN_ROWS, 128), lambda i: (i, 0)),
            pl.BlockSpec((BN_ROWS, 128), lambda i: (i, 0)),
        ],
        out_shape=[
            jax.ShapeDtypeStruct((N, 128), jnp.float32),
            jax.ShapeDtypeStruct((N, 128), jnp.float32),
            jax.ShapeDtypeStruct((N, 128), jnp.float32),
        ],
    )(feat, ntf, W_shared, b_shared, W2, scale, shift, W_het,
      W_self_shared, b_self_shared, W_self)


def _sc_agg(src, dst, q0, q1, zA, ones_rows):
    mesh = plsc.VectorSubcoreMesh(core_axis_name="c", subcore_axis_name="s")

    @functools.partial(
        pl.kernel,
        mesh=mesh,
        out_type=[
            jax.ShapeDtypeStruct((NC, N_PAD, D), jnp.float32),
            jax.ShapeDtypeStruct((NC, N_PAD, D), jnp.float32),
        ],
        scratch_types=[
            pltpu.VMEM((SB2 * CH,), jnp.int32),
            pltpu.VMEM((SB2 * CH,), jnp.int32),
            pltpu.VMEM((CH,), jnp.int32),
            pltpu.VMEM((CH,), jnp.int32),
            pltpu.VMEM((CH,), jnp.int32),
            pltpu.VMEM((CH,), jnp.int32),
            pltpu.VMEM((CH,), jnp.int32),
            pltpu.VMEM((CH, D), jnp.float32),
            pltpu.VMEM((CH, D), jnp.float32),
            pltpu.VMEM_SHARED((N_PAD, D), jnp.float32),
            pltpu.SemaphoreType.DMA,
            pltpu.SemaphoreType.DMA,
            pltpu.SemaphoreType.DMA,
            pltpu.SemaphoreType.DMA,
        ],
    )
    def agg(src_hbm, dst_hbm, q0_hbm, q1_hbm, zA_hbm, ones_hbm,
            outA_hbm, outD_hbm,
            srcb_v, dstb_v, dstc_v, srcc_v, dstr1_v, dstr2_v, dstr3_v,
            rows_a, rows_b, A_sp, sem_a, sem_b, sem_c, sem_d):
        cid = lax.axis_index("c")
        sid = lax.axis_index("s")
        rbase = sid * RPW

        def zero_acc():
            for b in range(RPW // CH):
                rb = rbase + b * CH
                pltpu.sync_copy(zA_hbm.at[pl.ds(rb, CH)], rows_a)
                pltpu.sync_copy(rows_a, A_sp.at[pl.ds(rb, CH)])

        def export(dst_out):
            for b in range(RPW // CH):
                rb = rbase + b * CH
                pltpu.sync_copy(A_sp.at[pl.ds(rb, CH)], rows_a)
                pltpu.sync_copy(rows_a, dst_out.at[cid, pl.ds(rb, CH)])

        def load_dstc(j, dst_ref=None):
            tgt = dstc_v if dst_ref is None else dst_ref
            for k in range(CH // L):
                tgt[pl.ds(k * L, L)] = dstb_v[pl.ds(j * CH + k * L, L)]

        zero_acc()
        plsc.subcore_barrier()

        cb1 = P1_BASE * sid + jnp.minimum(sid, P1_XTRA)

        def sweep(q_hbm):
            def sblock(sb, c):
                crow = cb1 + sb * SB1
                eb = crow * CH
                pltpu.sync_copy(src_hbm.at[pl.ds(eb, SB1 * CH)],
                                srcb_v.at[pl.ds(0, SB1 * CH)])
                pltpu.sync_copy(dst_hbm.at[pl.ds(eb, SB1 * CH)],
                                dstb_v.at[pl.ds(0, SB1 * CH)])
                cp = pltpu.async_copy(
                    q_hbm.at[srcb_v.at[pl.ds(0, CH)]], rows_a, sem_a)
                for j in range(SB1):
                    cur = rows_a if j % 2 == 0 else rows_b
                    nxt = rows_b if j % 2 == 0 else rows_a
                    nsem = sem_b if j % 2 == 0 else sem_a
                    load_dstc(j)
                    cp.wait()
                    if j + 1 < SB1:
                        cp = pltpu.async_copy(
                            q_hbm.at[srcb_v.at[pl.ds((j + 1) * CH, CH)]],
                            nxt, nsem)
                    pltpu.sync_copy(cur, A_sp.at[dstc_v], add=True)
                return c
            lax.fori_loop(0, NSB1, sblock, 0)

            @pl.when(sid < P1_XTRA)
            def _():
                eb = (cb1 + P1_BASE) * CH
                pltpu.sync_copy(src_hbm.at[pl.ds(eb, CH)], srcc_v)
                pltpu.sync_copy(dst_hbm.at[pl.ds(eb, CH)], dstc_v)
                pltpu.async_copy(q_hbm.at[srcc_v], rows_a, sem_a).wait()
                pltpu.sync_copy(rows_a, A_sp.at[dstc_v], add=True)

        @pl.when(cid == 0)
        def _():
            sweep(q0_hbm)

        @pl.when(cid == 1)
        def _():
            sweep(q1_hbm)

        plsc.subcore_barrier()
        export(outA_hbm)
        plsc.subcore_barrier()

        zero_acc()
        pltpu.sync_copy(ones_hbm, rows_b)
        plsc.subcore_barrier()

        cb2 = P2_CORE * cid + P2_BASE * sid + jnp.minimum(sid, P2_XTRA)

        ring_idx = [dstc_v, dstr1_v, dstr2_v, dstr3_v]
        ring_sem = [sem_a, sem_b, sem_c, sem_d]

        def sblock2(sb, c):
            eb = (cb2 + sb * SB2) * CH
            pltpu.sync_copy(dst_hbm.at[pl.ds(eb, SB2 * CH)], dstb_v)
            pending = [None] * 4
            for j in range(SB2):
                r = j % 4
                if pending[r] is not None:
                    pending[r].wait()
                load_dstc(j, ring_idx[r])
                pending[r] = pltpu.async_copy(
                    rows_b, A_sp.at[ring_idx[r]], ring_sem[r], add=True)
            for r in range(4):
                if pending[r] is not None:
                    pending[r].wait()
            return c
        lax.fori_loop(0, NSB2, sblock2, 0)

        @pl.when(sid < P2_XTRA)
        def _():
            eb = (cb2 + P2_BASE) * CH
            pltpu.sync_copy(dst_hbm.at[pl.ds(eb, CH)], dstc_v)
            pltpu.sync_copy(rows_b, A_sp.at[dstc_v], add=True)

        plsc.subcore_barrier()
        export(outD_hbm)

    return agg(src, dst, q0, q1, zA, ones_rows)


def _combine_body(A_ref, deg_ref, hself_ref, ntf_ref, bias_ref, out_ref):
    a = jnp.where(ntf_ref[...] == 0.0, A_ref[0], A_ref[1])
    d = deg_ref[0, :, 0:1] + deg_ref[1, :, 0:1]
    out_ref[...] = hself_ref[...] + a / jnp.maximum(d, 1.0) + bias_ref[...]


def _combine(A_part, deg_part, h_self, ntf, bias_row):
    nblk = N // BN_ROWS
    return pl.pallas_call(
        _combine_body,
        grid=(nblk,),
        in_specs=[
            pl.BlockSpec((2, BN_ROWS, 128), lambda i: (0, i, 0)),
            pl.BlockSpec((2, BN_ROWS, 128), lambda i: (0, i, 0)),
            pl.BlockSpec((BN_ROWS, 128), lambda i: (i, 0)),
            pl.BlockSpec((BN_ROWS, 1), lambda i: (i, 0)),
            pl.BlockSpec((1, 128), lambda i: (0, 0)),
        ],
        out_specs=pl.BlockSpec((BN_ROWS, 128), lambda i: (i, 0)),
        out_shape=jax.ShapeDtypeStruct((N, 128), jnp.float32),
    )(A_part, deg_part, h_self, ntf, bias_row)


def kernel(feat, edge_index, node_types, W_shared, b_shared, W2, bn_gamma,
           bn_beta, bn_mean, bn_var, W_het, W_self_shared, b_self_shared,
           W_self, bias_out):
    scale = (bn_gamma / jnp.sqrt(bn_var + 1e-5)).reshape(1, D)
    shift = (bn_beta - bn_mean * (bn_gamma / jnp.sqrt(bn_var + 1e-5))).reshape(1, D)
    ntf = node_types.astype(jnp.float32).reshape(N, 1)

    q0, q1, h_self = _prep(feat, ntf, W_shared, b_shared.reshape(1, D), W2,
                           scale, shift, W_het, W_self_shared,
                           b_self_shared.reshape(1, D), W_self)

    src = edge_index[0]
    dst = edge_index[1]
    zA = jnp.zeros((N_PAD, D), jnp.float32)
    ones_rows = jnp.ones((CH, D), jnp.float32)
    A_part, deg_part = _sc_agg(src, dst, q0, q1, zA, ones_rows)

    return _combine(A_part, deg_part, h_self, ntf, bias_out.reshape(1, D))


# speedup vs baseline: 28.3744x; 1.0900x over previous
import functools

import jax
import jax.numpy as jnp
from jax import lax
from jax.experimental import pallas as pl
from jax.experimental.pallas import tpu as pltpu
from jax.experimental.pallas import tpu_sc as plsc

N = 10000
E = 320000
D = 128
NC = 2
NS = 16
L = 16

N_PAD = 10240
RPW = N_PAD // NS
CH = 128
NCHUNK = E // CH
P1_BASE = NCHUNK // NS
P1_XTRA = NCHUNK - P1_BASE * NS
SB1 = 12
NSB1 = P1_BASE // SB1
P2_CORE = NCHUNK // NC
P2_BASE = P2_CORE // NS
P2_XTRA = P2_CORE - P2_BASE * NS
SB2 = 13
NSB2 = P2_BASE // SB2

BN_ROWS = 1000


def _prep_body(feat_ref, ntf_ref, Wsh_ref, bsh_ref, W2_ref, scale_ref,
               shift_ref, Whet_ref, Wss_ref, bss_ref, Wself_ref,
               q0_ref, q1_ref, hself_ref):
    f = feat_ref[...]
    t = jnp.dot(f, Wsh_ref[...], preferred_element_type=jnp.float32) + bsh_ref[...]
    h2 = jnp.dot(t, W2_ref[...], preferred_element_type=jnp.float32)
    h = jnp.maximum(h2 * scale_ref[...] + shift_ref[...], 0.0)
    m0 = ntf_ref[...] == 0.0
    p0 = jnp.dot(h, Whet_ref[0], preferred_element_type=jnp.float32)
    p1 = jnp.dot(h, Whet_ref[1], preferred_element_type=jnp.float32)
    p2 = jnp.dot(h, Whet_ref[2], preferred_element_type=jnp.float32)
    p3 = jnp.dot(h, Whet_ref[3], preferred_element_type=jnp.float32)
    q0_ref[...] = jnp.where(m0, p0, p2)
    q1_ref[...] = jnp.where(m0, p1, p3)
    hs = jnp.dot(f, Wss_ref[...], preferred_element_type=jnp.float32) + bss_ref[...]
    s0 = jnp.dot(hs, Wself_ref[0], preferred_element_type=jnp.float32)
    s1 = jnp.dot(hs, Wself_ref[1], preferred_element_type=jnp.float32)
    hself_ref[...] = jnp.where(m0, s0, s1)


def _prep(feat, ntf, W_shared, b_shared, W2, scale, shift, W_het,
          W_self_shared, b_self_shared, W_self):
    nblk = N // BN_ROWS
    full2 = pl.BlockSpec((128, 128), lambda i: (0, 0))
    row1 = pl.BlockSpec((1, 128), lambda i: (0, 0))
    return pl.pallas_call(
        _prep_body,
        grid=(nblk,),
        in_specs=[
            pl.BlockSpec((BN_ROWS, 128), lambda i: (i, 0)),
            pl.BlockSpec((BN_ROWS, 1), lambda i: (i, 0)),
            full2, row1, full2, row1, row1,
            pl.BlockSpec((4, 128, 128), lambda i: (0, 0, 0)),
            full2, row1,
            pl.BlockSpec((2, 128, 128), lambda i: (0, 0, 0)),
        ],
        out_specs=[
            pl.BlockSpec((BN_ROWS, 128), lambda i: (i, 0)),
            pl.BlockSpec((BN_ROWS, 128), lambda i: (i, 0)),
            pl.BlockSpec((BN_ROWS, 128), lambda i: (i, 0)),
        ],
        out_shape=[
            jax.ShapeDtypeStruct((N, 128), jnp.float32),
            jax.ShapeDtypeStruct((N, 128), jnp.float32),
            jax.ShapeDtypeStruct((N, 128), jnp.float32),
        ],
    )(feat, ntf, W_shared, b_shared, W2, scale, shift, W_het,
      W_self_shared, b_self_shared, W_self)


def _sc_agg(src, dst, q0, q1, zA, ones_rows):
    mesh = plsc.VectorSubcoreMesh(core_axis_name="c", subcore_axis_name="s")

    @functools.partial(
        pl.kernel,
        mesh=mesh,
        out_type=[
            jax.ShapeDtypeStruct((NC, N_PAD, D), jnp.float32),
            jax.ShapeDtypeStruct((NC, N_PAD, D), jnp.float32),
        ],
        scratch_types=[
            pltpu.VMEM((SB2 * CH,), jnp.int32),
            pltpu.VMEM((SB2 * CH,), jnp.int32),
            pltpu.VMEM((CH,), jnp.int32),
            pltpu.VMEM((CH,), jnp.int32),
            pltpu.VMEM((CH,), jnp.int32),
            pltpu.VMEM((CH,), jnp.int32),
            pltpu.VMEM((CH,), jnp.int32),
            pltpu.VMEM((CH, D), jnp.float32),
            pltpu.VMEM((CH, D), jnp.float32),
            pltpu.VMEM_SHARED((N_PAD, D), jnp.float32),
            pltpu.SemaphoreType.DMA,
            pltpu.SemaphoreType.DMA,
            pltpu.SemaphoreType.DMA,
            pltpu.SemaphoreType.DMA,
        ],
    )
    def agg(src_hbm, dst_hbm, q0_hbm, q1_hbm, zA_hbm, ones_hbm,
            outA_hbm, outD_hbm,
            srcb_v, dstb_v, dstc_v, srcc_v, dstr1_v, dstr2_v, dstr3_v,
            rows_a, rows_b, A_sp, sem_a, sem_b, sem_c, sem_d):
        cid = lax.axis_index("c")
        sid = lax.axis_index("s")
        rbase = sid * RPW

        def zero_acc():
            for b in range(RPW // CH):
                rb = rbase + b * CH
                pltpu.sync_copy(zA_hbm.at[pl.ds(rb, CH)], rows_a)
                pltpu.sync_copy(rows_a, A_sp.at[pl.ds(rb, CH)])

        def export(dst_out):
            for b in range(RPW // CH):
                rb = rbase + b * CH
                pltpu.sync_copy(A_sp.at[pl.ds(rb, CH)], rows_a)
                pltpu.sync_copy(rows_a, dst_out.at[cid, pl.ds(rb, CH)])

        def load_dstc(j, dst_ref=None):
            tgt = dstc_v if dst_ref is None else dst_ref
            for k in range(CH // L):
                tgt[pl.ds(k * L, L)] = dstb_v[pl.ds(j * CH + k * L, L)]

        zero_acc()
        plsc.subcore_barrier()

        cb1 = P1_BASE * sid + jnp.minimum(sid, P1_XTRA)

        def sweep(q_hbm):
            def sblock(sb, c):
                crow = cb1 + sb * SB1
                eb = crow * CH
                pltpu.sync_copy(src_hbm.at[pl.ds(eb, SB1 * CH)],
                                srcb_v.at[pl.ds(0, SB1 * CH)])
                pltpu.sync_copy(dst_hbm.at[pl.ds(eb, SB1 * CH)],
                                dstb_v.at[pl.ds(0, SB1 * CH)])
                cp = pltpu.async_copy(
                    q_hbm.at[srcb_v.at[pl.ds(0, CH)]], rows_a, sem_a)
                for j in range(SB1):
                    cur = rows_a if j % 2 == 0 else rows_b
                    nxt = rows_b if j % 2 == 0 else rows_a
                    nsem = sem_b if j % 2 == 0 else sem_a
                    load_dstc(j)
                    prev = cp
                    if j + 1 < SB1:
                        cp = pltpu.async_copy(
                            q_hbm.at[srcb_v.at[pl.ds((j + 1) * CH, CH)]],
                            nxt, nsem)
                    prev.wait()
                    pltpu.sync_copy(cur, A_sp.at[dstc_v], add=True)
                return c
            lax.fori_loop(0, NSB1, sblock, 0)

            @pl.when(sid < P1_XTRA)
            def _():
                eb = (cb1 + P1_BASE) * CH
                pltpu.sync_copy(src_hbm.at[pl.ds(eb, CH)], srcc_v)
                pltpu.sync_copy(dst_hbm.at[pl.ds(eb, CH)], dstc_v)
                pltpu.async_copy(q_hbm.at[srcc_v], rows_a, sem_a).wait()
                pltpu.sync_copy(rows_a, A_sp.at[dstc_v], add=True)

        @pl.when(cid == 0)
        def _():
            sweep(q0_hbm)

        @pl.when(cid == 1)
        def _():
            sweep(q1_hbm)

        plsc.subcore_barrier()
        export(outA_hbm)
        plsc.subcore_barrier()

        zero_acc()
        pltpu.sync_copy(ones_hbm, rows_b)
        plsc.subcore_barrier()

        cb2 = P2_CORE * cid + P2_BASE * sid + jnp.minimum(sid, P2_XTRA)

        ring_idx = [dstc_v, dstr1_v, dstr2_v, dstr3_v]
        ring_sem = [sem_a, sem_b, sem_c, sem_d]

        def sblock2(sb, c):
            eb = (cb2 + sb * SB2) * CH
            pltpu.sync_copy(dst_hbm.at[pl.ds(eb, SB2 * CH)], dstb_v)
            pending = [None] * 4
            for j in range(SB2):
                r = j % 4
                if pending[r] is not None:
                    pending[r].wait()
                load_dstc(j, ring_idx[r])
                pending[r] = pltpu.async_copy(
                    rows_b, A_sp.at[ring_idx[r]], ring_sem[r], add=True)
            for r in range(4):
                if pending[r] is not None:
                    pending[r].wait()
            return c
        lax.fori_loop(0, NSB2, sblock2, 0)

        @pl.when(sid < P2_XTRA)
        def _():
            eb = (cb2 + P2_BASE) * CH
            pltpu.sync_copy(dst_hbm.at[pl.ds(eb, CH)], dstc_v)
            pltpu.sync_copy(rows_b, A_sp.at[dstc_v], add=True)

        plsc.subcore_barrier()
        export(outD_hbm)

    return agg(src, dst, q0, q1, zA, ones_rows)


def _combine_body(A_ref, deg_ref, hself_ref, ntf_ref, bias_ref, out_ref):
    a = jnp.where(ntf_ref[...] == 0.0, A_ref[0], A_ref[1])
    d = deg_ref[0, :, 0:1] + deg_ref[1, :, 0:1]
    out_ref[...] = hself_ref[...] + a / jnp.maximum(d, 1.0) + bias_ref[...]


def _combine(A_part, deg_part, h_self, ntf, bias_row):
    nblk = N // BN_ROWS
    return pl.pallas_call(
        _combine_body,
        grid=(nblk,),
        in_specs=[
            pl.BlockSpec((2, BN_ROWS, 128), lambda i: (0, i, 0)),
            pl.BlockSpec((2, BN_ROWS, 128), lambda i: (0, i, 0)),
            pl.BlockSpec((BN_ROWS, 128), lambda i: (i, 0)),
            pl.BlockSpec((BN_ROWS, 1), lambda i: (i, 0)),
            pl.BlockSpec((1, 128), lambda i: (0, 0)),
        ],
        out_specs=pl.BlockSpec((BN_ROWS, 128), lambda i: (i, 0)),
        out_shape=jax.ShapeDtypeStruct((N, 128), jnp.float32),
    )(A_part, deg_part, h_self, ntf, bias_row)


def kernel(feat, edge_index, node_types, W_shared, b_shared, W2, bn_gamma,
           bn_beta, bn_mean, bn_var, W_het, W_self_shared, b_self_shared,
           W_self, bias_out):
    scale = (bn_gamma / jnp.sqrt(bn_var + 1e-5)).reshape(1, D)
    shift = (bn_beta - bn_mean * (bn_gamma / jnp.sqrt(bn_var + 1e-5))).reshape(1, D)
    ntf = node_types.astype(jnp.float32).reshape(N, 1)

    q0, q1, h_self = _prep(feat, ntf, W_shared, b_shared.reshape(1, D), W2,
                           scale, shift, W_het, W_self_shared,
                           b_self_shared.reshape(1, D), W_self)

    src = edge_index[0]
    dst = edge_index[1]
    zA = jnp.zeros((N_PAD, D), jnp.float32)
    ones_rows = jnp.ones((CH, D), jnp.float32)
    A_part, deg_part = _sc_agg(src, dst, q0, q1, zA, ones_rows)

    return _combine(A_part, deg_part, h_self, ntf, bias_out.reshape(1, D))
